# baseline (speedup 1.0000x reference)
import functools

import jax
import jax.numpy as jnp
from jax import lax
from jax.experimental import pallas as pl
from jax.experimental.pallas import tpu as pltpu
from jax.experimental.pallas import tpu_sc as plsc

_N = 10000
_E = 160000
_D = 256
_H = 256

_L = 100
_NS = 16
_NC = 2
_BN = 1000
_N2 = 10240
_NCH = 100
_NACC = _N2
_RPT = _NACC // _NS
_WD = 128
_G = 20
_NG = _NCH // _G
_NB = 3


_NW = _NC * _NS
_EPW = _E // _NW
_EPWP = 5008
_NV = _EPW // 16


def _deg_body(ei_hbm, zeros_hbm, out_hbm, dst_v, hist):
    c = lax.axis_index("c")
    t = lax.axis_index("s")
    w = c * _NS + t
    pltpu.sync_copy(ei_hbm.at[pl.ds(_E + w * _EPW, _EPW)],
                    dst_v.at[pl.ds(0, _EPW)])
    pltpu.sync_copy(zeros_hbm, hist)

    @pl.loop(0, _NV, unroll=4)
    def _(k):
        v = dst_v[pl.ds(k * 16, 16)]
        cnt, last = plsc.scan_count(v)
        plsc.addupdate_scatter(hist, [v], cnt, mask=last)

    tail = lax.iota(jnp.int32, 16) < (_EPW - _NV * 16)
    v = dst_v[pl.ds(_NV * 16, 16)]
    cnt, last = plsc.scan_count(v, mask=tail)
    plsc.addupdate_scatter(hist, [v], cnt, mask=last & tail)

    pltpu.sync_copy(hist, out_hbm.at[c, t])


_deg_call = functools.partial(
    pl.kernel,
    out_type=jax.ShapeDtypeStruct((_NC, _NS, _NACC), jnp.int32),
    mesh=plsc.VectorSubcoreMesh(core_axis_name="c", subcore_axis_name="s"),
    scratch_types=[
        pltpu.VMEM((_EPWP,), jnp.int32),
        pltpu.VMEM((_NACC,), jnp.int32),
    ],
    compiler_params=pltpu.CompilerParams(needs_layout_passes=False),
)(_deg_body)


def _scatter_body(tab_hbm, src_hbm, dst_hbm, zeros_hbm, out_hbm,
                  src_v, dst_v, rows_v, acc, gsem, ssem, zsem):
    c = lax.axis_index("c")
    t = lax.axis_index("s")
    tab_c = tab_hbm.at[c]
    base = t * _RPT

    def gstart(j):
        b = lax.rem(j, _NB)
        pltpu.async_copy(tab_c.at[src_v.at[j]], rows_v.at[b], gsem.at[b])

    def gwait(j):
        b = lax.rem(j, _NB)
        pltpu.make_async_copy(tab_c.at[src_v.at[j]], rows_v.at[b],
                              gsem.at[b]).wait()

    def sstart(j):
        b = lax.rem(j, _NB)
        pltpu.async_copy(rows_v.at[b], acc.at[dst_v.at[j]], ssem.at[b],
                         add=True)

    def swait(j):
        b = lax.rem(j, _NB)
        pltpu.make_async_copy(rows_v.at[b], acc.at[dst_v.at[j]],
                              ssem.at[b]).wait()

    @pl.when(t < _NS - 1)
    def _():
        for k in range(5):
            pltpu.async_copy(tab_c.at[pl.ds(base + k * 128, 128)],
                             acc.at[pl.ds(base + k * 128, 128)], zsem)
        for k in range(5):
            pltpu.make_async_copy(tab_c.at[pl.ds(base + k * 128, 128)],
                                  acc.at[pl.ds(base + k * 128, 128)],
                                  zsem).wait()

    @pl.when(t == _NS - 1)
    def _():
        cps = [
            (tab_c.at[pl.ds(base, 128)], acc.at[pl.ds(base, 128)]),
            (tab_c.at[pl.ds(base + 128, 128)], acc.at[pl.ds(base + 128, 128)]),
            (tab_c.at[pl.ds(base + 256, 128)], acc.at[pl.ds(base + 256, 128)]),
            (tab_c.at[pl.ds(base + 384, 16)], acc.at[pl.ds(base + 384, 16)]),
            (zeros_hbm, acc.at[pl.ds(_N, 128)]),
            (zeros_hbm.at[pl.ds(0, 112)], acc.at[pl.ds(_N + 128, 112)]),
        ]
        for s, d in cps:
            pltpu.async_copy(s, d, zsem)
        for s, d in cps:
            pltpu.make_async_copy(s, d, zsem).wait()

    plsc.subcore_barrier()

    for g in range(_NG):
        pltpu.sync_copy(src_hbm.at[t, g], src_v)
        pltpu.sync_copy(dst_hbm.at[t, g], dst_v)
        gstart(0)
        gstart(1)

        @pl.loop(0, _G)
        def _(j):
            gwait(j)
            sstart(j)

            @pl.when(j >= 1)
            def _():
                swait(jnp.maximum(j - 1, 0))

            @pl.when(j + 2 < _G)
            def _():
                gstart(jnp.minimum(j + 2, _G - 1))

        swait(_G - 1)

    plsc.subcore_barrier()
    pltpu.sync_copy(acc.at[pl.ds(base, _RPT)], out_hbm.at[c, pl.ds(base, _RPT)])


_scatter_call = functools.partial(
    pl.kernel,
    out_type=jax.ShapeDtypeStruct((_NC, _NACC, 128), jnp.float32),
    mesh=plsc.VectorSubcoreMesh(core_axis_name="c", subcore_axis_name="s"),
    scratch_types=[
        pltpu.VMEM((_G, _L), jnp.int32),
        pltpu.VMEM((_G, _L), jnp.int32),
        pltpu.VMEM((_NB, _L, 128), jnp.float32),
        pltpu.VMEM_SHARED((_NACC, 128), jnp.float32),
        pltpu.SemaphoreType.DMA((_NB,)),
        pltpu.SemaphoreType.DMA((_NB,)),
        pltpu.SemaphoreType.DMA,
    ],
)(_scatter_body)


def _dinv_body(deg_ref, dinv_ref):
    s = jnp.sum(deg_ref[...].astype(jnp.float32), axis=0) + 1.0
    d = lax.rsqrt(s)[:_N]
    dinv_ref[...] = jnp.broadcast_to(d[:, None], (_N, 8))


_dinv_call = pl.pallas_call(
    _dinv_body,
    in_specs=[pl.BlockSpec((_NW, _NACC), lambda: (0, 0))],
    out_specs=pl.BlockSpec((_N, 8), lambda: (0, 0)),
    out_shape=jax.ShapeDtypeStruct((_N, 8), jnp.float32),
)


def _g1_body(x_ref, w_ref, dinv_ref, g_ref):
    h = jnp.dot(x_ref[...], w_ref[...], preferred_element_type=jnp.float32,
                precision=lax.Precision.DEFAULT)
    g = h * dinv_ref[:, 0:1]
    g_ref[0, :, :] = g[:, :128]
    g_ref[1, :, :] = g[:, 128:]


_g1_call = pl.pallas_call(
    _g1_body,
    grid=(_N // _BN,),
    in_specs=[
        pl.BlockSpec((_BN, _D), lambda i: (i, 0)),
        pl.BlockSpec((_D, _H), lambda i: (0, 0)),
        pl.BlockSpec((_BN, 8), lambda i: (i, 0)),
    ],
    out_specs=pl.BlockSpec((_NC, _BN, 128), lambda i: (0, i, 0)),
    out_shape=jax.ShapeDtypeStruct((_NC, _N, 128), jnp.float32),
)


def _mid_body(acc_ref, dinv_ref, b_ref, w_ref, out_ref, *, elu):
    db = jnp.broadcast_to(dinv_ref[:, 0:1], (_BN, 128))
    b = b_ref[...]
    p0 = db * acc_ref[0, :, :] + b[:, :128]
    p1 = db * acc_ref[1, :, :] + b[:, 128:]
    p = jnp.concatenate([p0, p1], axis=1)
    if elu:
        x = jnp.where(p > 0.0, p, jnp.exp(jnp.minimum(p, 0.0)) - 1.0)
    else:
        x = jnp.maximum(p, 0.0)
    h = jnp.dot(x, w_ref[...], preferred_element_type=jnp.float32,
                precision=lax.Precision.DEFAULT)
    g = h * db[:, 0:1]
    out_ref[0, :, :] = g[:, :128]
    out_ref[1, :, :] = g[:, 128:]


def _make_mid(elu):
    return pl.pallas_call(
        functools.partial(_mid_body, elu=elu),
        grid=(_N // _BN,),
        in_specs=[
            pl.BlockSpec((_NC, _BN, 128), lambda i: (0, i, 0)),
            pl.BlockSpec((_BN, 8), lambda i: (i, 0)),
            pl.BlockSpec((1, _H), lambda i: (0, 0)),
            pl.BlockSpec((_H, _H), lambda i: (0, 0)),
        ],
        out_specs=pl.BlockSpec((_NC, _BN, 128), lambda i: (0, i, 0)),
        out_shape=jax.ShapeDtypeStruct((_NC, _N, 128), jnp.float32),
    )


_mid_elu = _make_mid(True)
_mid_relu = _make_mid(False)


def _final_body(acc_ref, dinv_ref, b_ref, out_ref):
    db = jnp.broadcast_to(dinv_ref[:, 0:1], (_BN, 128))
    b = b_ref[...]
    p0 = db * acc_ref[0, :, :] + b[:, :128]
    p1 = db * acc_ref[1, :, :] + b[:, 128:]
    out_ref[:, :128] = jnp.maximum(p0, 0.0)
    out_ref[:, 128:] = jnp.maximum(p1, 0.0)


_final_call = pl.pallas_call(
    _final_body,
    grid=(_N // _BN,),
    in_specs=[
        pl.BlockSpec((_NC, _BN, 128), lambda i: (0, i, 0)),
        pl.BlockSpec((_BN, 8), lambda i: (i, 0)),
        pl.BlockSpec((1, _H), lambda i: (0, 0)),
    ],
    out_specs=pl.BlockSpec((_BN, _H), lambda i: (i, 0)),
    out_shape=jax.ShapeDtypeStruct((_N, _H), jnp.float32),
)


def kernel(X, edge_index, W_in, b_in, W_hid, b_hid, W_iv, b_iv):
    src_h = edge_index[0].reshape(_NS, _NG, _G, _L)
    dst_h = edge_index[1].reshape(_NS, _NG, _G, _L)

    zacc = jnp.zeros((128, 128), jnp.float32)
    zhist = jnp.zeros((_NACC,), jnp.int32)
    b_in2 = b_in.reshape(1, _H)
    b_hid2 = b_hid.reshape(1, _H)
    b_iv2 = b_iv.reshape(1, _H)

    degp = _deg_call(edge_index.reshape(2 * _E), zhist)
    dinvb = _dinv_call(degp.reshape(_NW, _NACC))
    g1 = _g1_call(X, W_in, dinvb)
    acc1 = _scatter_call(g1, src_h, dst_h, zacc)
    g2 = _mid_elu(acc1, dinvb, b_in2, W_hid)
    acc2 = _scatter_call(g2, src_h, dst_h, zacc)
    g3 = _mid_relu(acc2, dinvb, b_hid2, W_iv)
    acc3 = _scatter_call(g3, src_h, dst_h, zacc)
    out = _final_call(acc3, dinvb, b_iv2)
    return out


# speedup vs baseline: 1.0023x; 1.0023x over previous
import functools

import jax
import jax.numpy as jnp
from jax import lax
from jax.experimental import pallas as pl
from jax.experimental.pallas import tpu as pltpu
from jax.experimental.pallas import tpu_sc as plsc

_N = 10000
_E = 160000
_D = 256
_H = 256

_L = 100
_NS = 16
_NC = 2
_BN = 1000
_N2 = 10240
_NCH = 100
_NACC = _N2
_RPT = _NACC // _NS
_G = 20
_NG = _NCH // _G
_NB = 3


_NW = _NC * _NS
_EPW = _E // _NW
_EPWP = 5008
_NV = _EPW // 16


def _deg_body(ei_hbm, zeros_hbm, out_hbm, dst_v, hist):
    c = lax.axis_index("c")
    t = lax.axis_index("s")
    w = c * _NS + t
    pltpu.sync_copy(ei_hbm.at[pl.ds(_E + w * _EPW, _EPW)],
                    dst_v.at[pl.ds(0, _EPW)])
    pltpu.sync_copy(zeros_hbm, hist)

    @pl.loop(0, _NV, unroll=4)
    def _(k):
        v = dst_v[pl.ds(k * 16, 16)]
        cnt, last = plsc.scan_count(v)
        plsc.addupdate_scatter(hist, [v], cnt, mask=last)

    tail = lax.iota(jnp.int32, 16) < (_EPW - _NV * 16)
    v = dst_v[pl.ds(_NV * 16, 16)]
    cnt, last = plsc.scan_count(v, mask=tail)
    plsc.addupdate_scatter(hist, [v], cnt, mask=last & tail)

    pltpu.sync_copy(hist, out_hbm.at[c, t])


_deg_call = functools.partial(
    pl.kernel,
    out_type=jax.ShapeDtypeStruct((_NC, _NS, _NACC), jnp.int32),
    mesh=plsc.VectorSubcoreMesh(core_axis_name="c", subcore_axis_name="s"),
    scratch_types=[
        pltpu.VMEM((_EPWP,), jnp.int32),
        pltpu.VMEM((_NACC,), jnp.int32),
    ],
    compiler_params=pltpu.CompilerParams(needs_layout_passes=False),
)(_deg_body)


def _scatter_body(tab_hbm, src_hbm, dst_hbm, zeros_hbm, out_hbm,
                  src_v, dst_v, rows_v, acc, gsem, ssem, zsem):
    c = lax.axis_index("c")
    t = lax.axis_index("s")
    tab_c = tab_hbm.at[c]
    base = t * _RPT

    def gstart(j):
        b = lax.rem(j, _NB)
        pltpu.async_copy(tab_c.at[src_v.at[j]], rows_v.at[b], gsem.at[b])

    def gwait(j):
        b = lax.rem(j, _NB)
        pltpu.make_async_copy(tab_c.at[src_v.at[j]], rows_v.at[b],
                              gsem.at[b]).wait()

    def sstart(j):
        b = lax.rem(j, _NB)
        pltpu.async_copy(rows_v.at[b], acc.at[dst_v.at[j]], ssem.at[b],
                         add=True)

    def swait(j):
        b = lax.rem(j, _NB)
        pltpu.make_async_copy(rows_v.at[b], acc.at[dst_v.at[j]],
                              ssem.at[b]).wait()

    @pl.when(t < _NS - 1)
    def _():
        for k in range(5):
            pltpu.async_copy(tab_c.at[pl.ds(base + k * 128, 128)],
                             acc.at[pl.ds(base + k * 128, 128)], zsem)
        for k in range(5):
            pltpu.make_async_copy(tab_c.at[pl.ds(base + k * 128, 128)],
                                  acc.at[pl.ds(base + k * 128, 128)],
                                  zsem).wait()

    @pl.when(t == _NS - 1)
    def _():
        cps = [
            (tab_c.at[pl.ds(base, 128)], acc.at[pl.ds(base, 128)]),
            (tab_c.at[pl.ds(base + 128, 128)], acc.at[pl.ds(base + 128, 128)]),
            (tab_c.at[pl.ds(base + 256, 128)], acc.at[pl.ds(base + 256, 128)]),
            (tab_c.at[pl.ds(base + 384, 16)], acc.at[pl.ds(base + 384, 16)]),
            (zeros_hbm, acc.at[pl.ds(_N, 128)]),
            (zeros_hbm.at[pl.ds(0, 112)], acc.at[pl.ds(_N + 128, 112)]),
        ]
        for s, d in cps:
            pltpu.async_copy(s, d, zsem)
        for s, d in cps:
            pltpu.make_async_copy(s, d, zsem).wait()

    plsc.subcore_barrier()

    for g in range(_NG):
        pltpu.sync_copy(src_hbm.at[t, g], src_v)
        pltpu.sync_copy(dst_hbm.at[t, g], dst_v)
        gstart(0)
        gstart(1)

        @pl.loop(0, _G)
        def _(j):
            gwait(j)
            sstart(j)

            @pl.when(j >= 1)
            def _():
                swait(jnp.maximum(j - 1, 0))

            @pl.when(j + 2 < _G)
            def _():
                gstart(jnp.minimum(j + 2, _G - 1))

        swait(_G - 1)

    plsc.subcore_barrier()
    pltpu.sync_copy(acc.at[pl.ds(base, _RPT)], out_hbm.at[c, pl.ds(base, _RPT)])


_scatter_call = functools.partial(
    pl.kernel,
    out_type=jax.ShapeDtypeStruct((_NC, _NACC, 128), jnp.float32),
    mesh=plsc.VectorSubcoreMesh(core_axis_name="c", subcore_axis_name="s"),
    scratch_types=[
        pltpu.VMEM((_G, _L), jnp.int32),
        pltpu.VMEM((_G, _L), jnp.int32),
        pltpu.VMEM((_NB, _L, 128), jnp.float32),
        pltpu.VMEM_SHARED((_NACC, 128), jnp.float32),
        pltpu.SemaphoreType.DMA((_NB,)),
        pltpu.SemaphoreType.DMA((_NB,)),
        pltpu.SemaphoreType.DMA,
    ],
)(_scatter_body)


def _dinv_body(deg_ref, dinv_ref):
    s = jnp.sum(deg_ref[...].astype(jnp.float32), axis=0) + 1.0
    d = lax.rsqrt(s)[:_N]
    dinv_ref[...] = jnp.broadcast_to(d[:, None], (_N, 8))


_dinv_call = pl.pallas_call(
    _dinv_body,
    in_specs=[pl.BlockSpec((_NW, _NACC), lambda: (0, 0))],
    out_specs=pl.BlockSpec((_N, 8), lambda: (0, 0)),
    out_shape=jax.ShapeDtypeStruct((_N, 8), jnp.float32),
)


def _g1_body(x_ref, w_ref, dinv_ref, g_ref):
    h = jnp.dot(x_ref[...], w_ref[...], preferred_element_type=jnp.float32)
    g = h * dinv_ref[:, 0:1]
    g_ref[0, :, :] = g[:, :128]
    g_ref[1, :, :] = g[:, 128:]


_g1_call = pl.pallas_call(
    _g1_body,
    grid=(_N // _BN,),
    in_specs=[
        pl.BlockSpec((_BN, _D), lambda i: (i, 0)),
        pl.BlockSpec((_D, _H), lambda i: (0, 0)),
        pl.BlockSpec((_BN, 8), lambda i: (i, 0)),
    ],
    out_specs=pl.BlockSpec((_NC, _BN, 128), lambda i: (0, i, 0)),
    out_shape=jax.ShapeDtypeStruct((_NC, _N, 128), jnp.float32),
)


def _mid_body(acc_ref, dinv_ref, b_ref, w_ref, out_ref, *, elu):
    db = jnp.broadcast_to(dinv_ref[:, 0:1], (_BN, 128))
    b = b_ref[...]
    p0 = db * acc_ref[0, :, :] + b[:, :128]
    p1 = db * acc_ref[1, :, :] + b[:, 128:]
    p = jnp.concatenate([p0, p1], axis=1)
    if elu:
        x = jnp.where(p > 0.0, p, jnp.exp(jnp.minimum(p, 0.0)) - 1.0)
    else:
        x = jnp.maximum(p, 0.0)
    h = jnp.dot(x, w_ref[...], preferred_element_type=jnp.float32)
    g = h * db[:, 0:1]
    out_ref[0, :, :] = g[:, :128]
    out_ref[1, :, :] = g[:, 128:]


def _make_mid(elu):
    return pl.pallas_call(
        functools.partial(_mid_body, elu=elu),
        grid=(_N // _BN,),
        in_specs=[
            pl.BlockSpec((_NC, _BN, 128), lambda i: (0, i, 0)),
            pl.BlockSpec((_BN, 8), lambda i: (i, 0)),
            pl.BlockSpec((1, _H), lambda i: (0, 0)),
            pl.BlockSpec((_H, _H), lambda i: (0, 0)),
        ],
        out_specs=pl.BlockSpec((_NC, _BN, 128), lambda i: (0, i, 0)),
        out_shape=jax.ShapeDtypeStruct((_NC, _N, 128), jnp.float32),
    )


_mid_elu = _make_mid(True)
_mid_relu = _make_mid(False)


def _final_body(acc_ref, dinv_ref, b_ref, out_ref):
    db = jnp.broadcast_to(dinv_ref[:, 0:1], (_BN, 128))
    b = b_ref[...]
    p0 = db * acc_ref[0, :, :] + b[:, :128]
    p1 = db * acc_ref[1, :, :] + b[:, 128:]
    out_ref[:, :128] = jnp.maximum(p0, 0.0)
    out_ref[:, 128:] = jnp.maximum(p1, 0.0)


_final_call = pl.pallas_call(
    _final_body,
    grid=(_N // _BN,),
    in_specs=[
        pl.BlockSpec((_NC, _BN, 128), lambda i: (0, i, 0)),
        pl.BlockSpec((_BN, 8), lambda i: (i, 0)),
        pl.BlockSpec((1, _H), lambda i: (0, 0)),
    ],
    out_specs=pl.BlockSpec((_BN, _H), lambda i: (i, 0)),
    out_shape=jax.ShapeDtypeStruct((_N, _H), jnp.float32),
)


def kernel(X, edge_index, W_in, b_in, W_hid, b_hid, W_iv, b_iv):
    src_h = edge_index[0].reshape(_NS, _NG, _G, _L)
    dst_h = edge_index[1].reshape(_NS, _NG, _G, _L)

    zacc = jnp.zeros((128, 128), jnp.float32)
    zhist = jnp.zeros((_NACC,), jnp.int32)
    b_in2 = b_in.reshape(1, _H)
    b_hid2 = b_hid.reshape(1, _H)
    b_iv2 = b_iv.reshape(1, _H)

    degp = _deg_call(edge_index.reshape(2 * _E), zhist)
    dinvb = _dinv_call(degp.reshape(_NW, _NACC))
    g1 = _g1_call(X, W_in, dinvb)
    acc1 = _scatter_call(g1, src_h, dst_h, zacc)
    g2 = _mid_elu(acc1, dinvb, b_in2, W_hid)
    acc2 = _scatter_call(g2, src_h, dst_h, zacc)
    g3 = _mid_relu(acc2, dinvb, b_hid2, W_iv)
    acc3 = _scatter_call(g3, src_h, dst_h, zacc)
    out = _final_call(acc3, dinvb, b_iv2)
    return out


# speedup vs baseline: 1.0199x; 1.0176x over previous
import functools

import jax
import jax.numpy as jnp
from jax import lax
from jax.experimental import pallas as pl
from jax.experimental.pallas import tpu as pltpu
from jax.experimental.pallas import tpu_sc as plsc

_N = 10000
_E = 160000
_D = 256
_H = 256

_L = 100
_NS = 16
_NC = 2
_BN = 1000
_N2 = 10240
_NCH = 100
_NACC = _N2
_RPT = _NACC // _NS
_G = 25
_NG = _NCH // _G
_NB = 3


_NW = _NC * _NS
_EPW = _E // _NW
_EPWP = 5008
_NV = _EPW // 16


def _deg_body(ei_hbm, zeros_hbm, out_hbm, dst_v, hist):
    c = lax.axis_index("c")
    t = lax.axis_index("s")
    w = c * _NS + t
    pltpu.sync_copy(ei_hbm.at[pl.ds(_E + w * _EPW, _EPW)],
                    dst_v.at[pl.ds(0, _EPW)])
    pltpu.sync_copy(zeros_hbm, hist)

    @pl.loop(0, _NV, unroll=4)
    def _(k):
        v = dst_v[pl.ds(k * 16, 16)]
        cnt, last = plsc.scan_count(v)
        plsc.addupdate_scatter(hist, [v], cnt, mask=last)

    tail = lax.iota(jnp.int32, 16) < (_EPW - _NV * 16)
    v = dst_v[pl.ds(_NV * 16, 16)]
    cnt, last = plsc.scan_count(v, mask=tail)
    plsc.addupdate_scatter(hist, [v], cnt, mask=last & tail)

    pltpu.sync_copy(hist, out_hbm.at[c, t])


_deg_call = functools.partial(
    pl.kernel,
    out_type=jax.ShapeDtypeStruct((_NC, _NS, _NACC), jnp.int32),
    mesh=plsc.VectorSubcoreMesh(core_axis_name="c", subcore_axis_name="s"),
    scratch_types=[
        pltpu.VMEM((_EPWP,), jnp.int32),
        pltpu.VMEM((_NACC,), jnp.int32),
    ],
    compiler_params=pltpu.CompilerParams(needs_layout_passes=False),
)(_deg_body)


def _scatter_body(tab_hbm, src_hbm, dst_hbm, zeros_hbm, out_hbm,
                  src_v, dst_v, rows_v, acc, gsem, ssem, zsem):
    c = lax.axis_index("c")
    t = lax.axis_index("s")
    tab_c = tab_hbm.at[c]
    base = t * _RPT

    def gstart(j):
        b = lax.rem(j, _NB)
        pltpu.async_copy(tab_c.at[src_v.at[j]], rows_v.at[b], gsem.at[b])

    def gwait(j):
        b = lax.rem(j, _NB)
        pltpu.make_async_copy(tab_c.at[src_v.at[j]], rows_v.at[b],
                              gsem.at[b]).wait()

    def sstart(j):
        b = lax.rem(j, _NB)
        pltpu.async_copy(rows_v.at[b], acc.at[dst_v.at[j]], ssem.at[b],
                         add=True)

    def swait(j):
        b = lax.rem(j, _NB)
        pltpu.make_async_copy(rows_v.at[b], acc.at[dst_v.at[j]],
                              ssem.at[b]).wait()

    @pl.when(t < _NS - 1)
    def _():
        for k in range(5):
            pltpu.async_copy(tab_c.at[pl.ds(base + k * 128, 128)],
                             acc.at[pl.ds(base + k * 128, 128)], zsem)
        for k in range(5):
            pltpu.make_async_copy(tab_c.at[pl.ds(base + k * 128, 128)],
                                  acc.at[pl.ds(base + k * 128, 128)],
                                  zsem).wait()

    @pl.when(t == _NS - 1)
    def _():
        cps = [
            (tab_c.at[pl.ds(base, 128)], acc.at[pl.ds(base, 128)]),
            (tab_c.at[pl.ds(base + 128, 128)], acc.at[pl.ds(base + 128, 128)]),
            (tab_c.at[pl.ds(base + 256, 128)], acc.at[pl.ds(base + 256, 128)]),
            (tab_c.at[pl.ds(base + 384, 16)], acc.at[pl.ds(base + 384, 16)]),
            (zeros_hbm, acc.at[pl.ds(_N, 128)]),
            (zeros_hbm.at[pl.ds(0, 112)], acc.at[pl.ds(_N + 128, 112)]),
        ]
        for s, d in cps:
            pltpu.async_copy(s, d, zsem)
        for s, d in cps:
            pltpu.make_async_copy(s, d, zsem).wait()

    plsc.subcore_barrier()

    for g in range(_NG):
        pltpu.sync_copy(src_hbm.at[t, g], src_v)
        pltpu.sync_copy(dst_hbm.at[t, g], dst_v)
        gstart(0)
        gstart(1)

        @pl.loop(0, _G)
        def _(j):
            gwait(j)
            sstart(j)

            @pl.when(j >= 1)
            def _():
                swait(jnp.maximum(j - 1, 0))

            @pl.when(j + 2 < _G)
            def _():
                gstart(jnp.minimum(j + 2, _G - 1))

        swait(_G - 1)

    plsc.subcore_barrier()
    pltpu.sync_copy(acc.at[pl.ds(base, _RPT)], out_hbm.at[c, pl.ds(base, _RPT)])


_scatter_call = functools.partial(
    pl.kernel,
    out_type=jax.ShapeDtypeStruct((_NC, _NACC, 128), jnp.float32),
    mesh=plsc.VectorSubcoreMesh(core_axis_name="c", subcore_axis_name="s"),
    scratch_types=[
        pltpu.VMEM((_G, _L), jnp.int32),
        pltpu.VMEM((_G, _L), jnp.int32),
        pltpu.VMEM((_NB, _L, 128), jnp.float32),
        pltpu.VMEM_SHARED((_NACC, 128), jnp.float32),
        pltpu.SemaphoreType.DMA((_NB,)),
        pltpu.SemaphoreType.DMA((_NB,)),
        pltpu.SemaphoreType.DMA,
    ],
)(_scatter_body)


def _dinv_body(deg_ref, dinv_ref):
    s = jnp.sum(deg_ref[...].astype(jnp.float32), axis=0) + 1.0
    d = lax.rsqrt(s)[:_N]
    dinv_ref[...] = jnp.broadcast_to(d[:, None], (_N, 8))


_dinv_call = pl.pallas_call(
    _dinv_body,
    in_specs=[pl.BlockSpec((_NW, _NACC), lambda: (0, 0))],
    out_specs=pl.BlockSpec((_N, 8), lambda: (0, 0)),
    out_shape=jax.ShapeDtypeStruct((_N, 8), jnp.float32),
)


def _g1_body(x_ref, w_ref, dinv_ref, g_ref):
    h = jnp.dot(x_ref[...], w_ref[...], preferred_element_type=jnp.float32)
    g = h * dinv_ref[:, 0:1]
    g_ref[0, :, :] = g[:, :128]
    g_ref[1, :, :] = g[:, 128:]


_g1_call = pl.pallas_call(
    _g1_body,
    grid=(_N // _BN,),
    in_specs=[
        pl.BlockSpec((_BN, _D), lambda i: (i, 0)),
        pl.BlockSpec((_D, _H), lambda i: (0, 0)),
        pl.BlockSpec((_BN, 8), lambda i: (i, 0)),
    ],
    out_specs=pl.BlockSpec((_NC, _BN, 128), lambda i: (0, i, 0)),
    out_shape=jax.ShapeDtypeStruct((_NC, _N, 128), jnp.float32),
)


def _mid_body(acc_ref, dinv_ref, b_ref, w_ref, out_ref, *, elu):
    db = jnp.broadcast_to(dinv_ref[:, 0:1], (_BN, 128))
    b = b_ref[...]
    p0 = db * acc_ref[0, :, :] + b[:, :128]
    p1 = db * acc_ref[1, :, :] + b[:, 128:]
    p = jnp.concatenate([p0, p1], axis=1)
    if elu:
        x = jnp.where(p > 0.0, p, jnp.exp(jnp.minimum(p, 0.0)) - 1.0)
    else:
        x = jnp.maximum(p, 0.0)
    h = jnp.dot(x, w_ref[...], preferred_element_type=jnp.float32)
    g = h * db[:, 0:1]
    out_ref[0, :, :] = g[:, :128]
    out_ref[1, :, :] = g[:, 128:]


def _make_mid(elu):
    return pl.pallas_call(
        functools.partial(_mid_body, elu=elu),
        grid=(_N // _BN,),
        in_specs=[
            pl.BlockSpec((_NC, _BN, 128), lambda i: (0, i, 0)),
            pl.BlockSpec((_BN, 8), lambda i: (i, 0)),
            pl.BlockSpec((1, _H), lambda i: (0, 0)),
            pl.BlockSpec((_H, _H), lambda i: (0, 0)),
        ],
        out_specs=pl.BlockSpec((_NC, _BN, 128), lambda i: (0, i, 0)),
        out_shape=jax.ShapeDtypeStruct((_NC, _N, 128), jnp.float32),
    )


_mid_elu = _make_mid(True)
_mid_relu = _make_mid(False)


def _final_body(acc_ref, dinv_ref, b_ref, out_ref):
    db = jnp.broadcast_to(dinv_ref[:, 0:1], (_BN, 128))
    b = b_ref[...]
    p0 = db * acc_ref[0, :, :] + b[:, :128]
    p1 = db * acc_ref[1, :, :] + b[:, 128:]
    out_ref[:, :128] = jnp.maximum(p0, 0.0)
    out_ref[:, 128:] = jnp.maximum(p1, 0.0)


_final_call = pl.pallas_call(
    _final_body,
    grid=(_N // _BN,),
    in_specs=[
        pl.BlockSpec((_NC, _BN, 128), lambda i: (0, i, 0)),
        pl.BlockSpec((_BN, 8), lambda i: (i, 0)),
        pl.BlockSpec((1, _H), lambda i: (0, 0)),
    ],
    out_specs=pl.BlockSpec((_BN, _H), lambda i: (i, 0)),
    out_shape=jax.ShapeDtypeStruct((_N, _H), jnp.float32),
)


def kernel(X, edge_index, W_in, b_in, W_hid, b_hid, W_iv, b_iv):
    src_h = edge_index[0].reshape(_NS, _NG, _G, _L)
    dst_h = edge_index[1].reshape(_NS, _NG, _G, _L)

    zacc = jnp.zeros((128, 128), jnp.float32)
    zhist = jnp.zeros((_NACC,), jnp.int32)
    b_in2 = b_in.reshape(1, _H)
    b_hid2 = b_hid.reshape(1, _H)
    b_iv2 = b_iv.reshape(1, _H)

    degp = _deg_call(edge_index.reshape(2 * _E), zhist)
    dinvb = _dinv_call(degp.reshape(_NW, _NACC))
    g1 = _g1_call(X, W_in, dinvb)
    acc1 = _scatter_call(g1, src_h, dst_h, zacc)
    g2 = _mid_elu(acc1, dinvb, b_in2, W_hid)
    acc2 = _scatter_call(g2, src_h, dst_h, zacc)
    g3 = _mid_relu(acc2, dinvb, b_hid2, W_iv)
    acc3 = _scatter_call(g3, src_h, dst_h, zacc)
    out = _final_call(acc3, dinvb, b_iv2)
    return out
